# baseline (speedup 1.0000x reference)
import jax
import jax.numpy as jnp
from jax import lax
from jax.experimental import pallas as pl
from jax.experimental.pallas import tpu as pltpu

_TEMP = 0.2
_L1 = 1.0
_L2 = 0.2
_L3 = 1e-07
_D = 128
_BLK = 512
_B = 4096


def _layer_body(msg_ref, prev_ref, intents_ref, w_ref, b_ref, out_ref):
    x = msg_ref[...]
    intents = intents_ref[...]
    s = lax.dot_general(x, intents, (((1,), (1,)), ((), ())),
                        preferred_element_type=jnp.float32) * (1.0 / _TEMP)
    m = jnp.max(s, axis=1, keepdims=True)
    e = jnp.exp(s - m)
    p = e / jnp.sum(e, axis=1, keepdims=True)
    xe = jnp.dot(p, intents, preferred_element_type=jnp.float32)
    t = lax.dot_general(xe, w_ref[...], (((1,), (1,)), ((), ())),
                        preferred_element_type=jnp.float32) + b_ref[...]
    r = t + prev_ref[...]
    mu = jnp.mean(r, axis=1, keepdims=True)
    var = jnp.mean((r - mu) * (r - mu), axis=1, keepdims=True)
    out_ref[...] = (r - mu) * lax.rsqrt(var + 1e-5)


def _layer_update(msg, prev, intents, w, b):
    n = msg.shape[0]
    grid = (n + _BLK - 1) // _BLK
    return pl.pallas_call(
        _layer_body,
        grid=(grid,),
        in_specs=[
            pl.BlockSpec((_BLK, _D), lambda i: (i, 0)),
            pl.BlockSpec((_BLK, _D), lambda i: (i, 0)),
            pl.BlockSpec((_D, _D), lambda i: (0, 0)),
            pl.BlockSpec((_D, _D), lambda i: (0, 0)),
            pl.BlockSpec((1, _D), lambda i: (0, 0)),
        ],
        out_specs=pl.BlockSpec((_BLK, _D), lambda i: (i, 0)),
        out_shape=jax.ShapeDtypeStruct((n, _D), jnp.float32),
    )(msg, prev, intents, w, b)


def _loss_body(ua_ref, ia_ref, u_ref, p_ref, n_ref, intents_ref,
               bpr_ref, con_ref, reg_ref):
    k = pl.program_id(0)

    @pl.when(k == 0)
    def _init():
        u = u_ref[...]
        pe = p_ref[...]
        ne = n_ref[...]
        pos = jnp.sum(u * pe, axis=1)
        neg = jnp.sum(u * ne, axis=1)
        d = pos - neg
        sig = 1.0 / (1.0 + jnp.exp(-d))
        bpr_ref[0, 0] = -jnp.sum(jnp.log(sig + 1e-10))
        ints = intents_ref[...]
        reg_ref[0, 0] = (jnp.sqrt(jnp.sum(u * u)) + jnp.sqrt(jnp.sum(pe * pe))
                        + jnp.sqrt(jnp.sum(ne * ne))
                        + jnp.sqrt(jnp.sum(ints * ints)))
        con_ref[0, 0] = 0.0

    ua = ua_ref[...]
    sim = lax.dot_general(ua, ia_ref[...], (((1,), (1,)), ((), ())),
                          preferred_element_type=jnp.float32) * (1.0 / _TEMP)
    m = jnp.max(sim, axis=1, keepdims=True)
    lse = m[:, 0] + jnp.log(jnp.sum(jnp.exp(sim - m), axis=1))
    ii = lax.broadcasted_iota(jnp.int32, (_BLK, _B), 0)
    jj = lax.broadcasted_iota(jnp.int32, (_BLK, _B), 1)
    diag = jnp.sum(jnp.where(jj == ii + k * _BLK, sim, 0.0), axis=1)
    con_ref[0, 0] += jnp.sum(lse - diag)


def _loss_call(ua, ia, u_emb, p_emb, n_emb, intents):
    grid = _B // _BLK
    scalar = jax.ShapeDtypeStruct((1, 1), jnp.float32)
    full = lambda shape: pl.BlockSpec(shape, lambda i: (0, 0))
    return pl.pallas_call(
        _loss_body,
        grid=(grid,),
        in_specs=[
            pl.BlockSpec((_BLK, _D), lambda i: (i, 0)),
            full((_B, _D)),
            full((_B, _D)),
            full((_B, _D)),
            full((_B, _D)),
            full((_D, _D)),
        ],
        out_specs=[pl.BlockSpec(memory_space=pltpu.SMEM)] * 3,
        out_shape=[scalar, scalar, scalar],
    )(ua, ia, u_emb, p_emb, n_emb, intents)


def _spmm(rows, cols, vals, mat, n_out):
    gathered = jnp.take(mat, cols, axis=0) * vals[:, None]
    return jax.ops.segment_sum(gathered, rows, num_segments=n_out)


def kernel(user_embedding, item_embedding, intents, W_w, W_b, u_mul_s,
           v_mul_s, ut, vt, adj_vals, adj_rows, adj_cols, user_ids,
           item_ids, pos_items, neg_items):
    n_users = user_embedding.shape[0]
    n_items = item_embedding.shape[0]
    ue, ie = user_embedding, item_embedding
    ue_sum, ie_sum = ue, ie
    msgu_sum = None
    msgi_sum = None
    n_layers = W_w.shape[0]
    for l in range(n_layers):
        mu = _spmm(adj_rows, adj_cols, adj_vals, ie, n_users)
        mi = _spmm(adj_cols, adj_rows, adj_vals, ue, n_items)
        msgu_sum = mu if msgu_sum is None else msgu_sum + mu
        msgi_sum = mi if msgi_sum is None else msgi_sum + mi
        b = W_b[l][None, :]
        ue_new = _layer_update(mu, ue, intents, W_w[l], b)
        ie_new = _layer_update(mi, ie, intents, W_w[l], b)
        ue, ie = ue_new, ie_new
        ue_sum = ue_sum + ue
        ie_sum = ie_sum + ie
    inv = 1.0 / (n_layers + 1)
    final_u = ue_sum * inv
    final_i = ie_sum * inv
    u_emb = jnp.take(final_u, user_ids, axis=0)
    p_emb = jnp.take(final_i, pos_items, axis=0)
    n_emb = jnp.take(final_i, neg_items, axis=0)
    ua = jnp.take(msgu_sum * (1.0 / n_layers), user_ids, axis=0)
    ia = jnp.take(msgi_sum * (1.0 / n_layers), item_ids, axis=0)
    bpr_s, con_s, reg_s = _loss_call(ua, ia, u_emb, p_emb, n_emb, intents)
    loss_bpr = bpr_s[0, 0] / _B
    loss_contrast = con_s[0, 0] / _B
    reg = _L3 * reg_s[0, 0]
    loss = _L1 * loss_bpr + _L2 * loss_contrast + reg
    return (loss, loss_bpr, loss_contrast, reg)


# speedup vs baseline: 1.0100x; 1.0100x over previous
import jax
import jax.numpy as jnp
from jax import lax
from jax.experimental import pallas as pl
from jax.experimental.pallas import tpu as pltpu

_TEMP = 0.2
_L1 = 1.0
_L2 = 0.2
_L3 = 1e-07
_D = 128
_BLK = 512
_LBLK = 2000
_B = 4096


def _layer_body(msg_ref, prev_ref, intents_ref, w_ref, b_ref, out_ref):
    x = msg_ref[...]
    intents = intents_ref[...]
    s = lax.dot_general(x, intents, (((1,), (1,)), ((), ())),
                        preferred_element_type=jnp.float32) * (1.0 / _TEMP)
    m = jnp.max(s, axis=1, keepdims=True)
    e = jnp.exp(s - m)
    p = e / jnp.sum(e, axis=1, keepdims=True)
    xe = jnp.dot(p, intents, preferred_element_type=jnp.float32)
    t = lax.dot_general(xe, w_ref[...], (((1,), (1,)), ((), ())),
                        preferred_element_type=jnp.float32) + b_ref[...]
    r = t + prev_ref[...]
    mu = jnp.mean(r, axis=1, keepdims=True)
    var = jnp.mean((r - mu) * (r - mu), axis=1, keepdims=True)
    out_ref[...] = (r - mu) * lax.rsqrt(var + 1e-5)


def _layer_update(msg, prev, intents, w, b):
    n = msg.shape[0]
    grid = (n + _LBLK - 1) // _LBLK
    return pl.pallas_call(
        _layer_body,
        grid=(grid,),
        in_specs=[
            pl.BlockSpec((_LBLK, _D), lambda i: (i, 0)),
            pl.BlockSpec((_LBLK, _D), lambda i: (i, 0)),
            pl.BlockSpec((_D, _D), lambda i: (0, 0)),
            pl.BlockSpec((_D, _D), lambda i: (0, 0)),
            pl.BlockSpec((1, _D), lambda i: (0, 0)),
        ],
        out_specs=pl.BlockSpec((_LBLK, _D), lambda i: (i, 0)),
        out_shape=jax.ShapeDtypeStruct((n, _D), jnp.float32),
    )(msg, prev, intents, w, b)


def _loss_body(ua_ref, ia_ref, u_ref, p_ref, n_ref, intents_ref,
               bpr_ref, con_ref, reg_ref):
    k = pl.program_id(0)

    @pl.when(k == 0)
    def _init():
        u = u_ref[...]
        pe = p_ref[...]
        ne = n_ref[...]
        pos = jnp.sum(u * pe, axis=1)
        neg = jnp.sum(u * ne, axis=1)
        d = pos - neg
        sig = 1.0 / (1.0 + jnp.exp(-d))
        bpr_ref[0, 0] = -jnp.sum(jnp.log(sig + 1e-10))
        ints = intents_ref[...]
        reg_ref[0, 0] = (jnp.sqrt(jnp.sum(u * u)) + jnp.sqrt(jnp.sum(pe * pe))
                        + jnp.sqrt(jnp.sum(ne * ne))
                        + jnp.sqrt(jnp.sum(ints * ints)))
        con_ref[0, 0] = 0.0

    ua = ua_ref[...]
    sim = lax.dot_general(ua, ia_ref[...], (((1,), (1,)), ((), ())),
                          preferred_element_type=jnp.float32) * (1.0 / _TEMP)
    m = jnp.max(sim, axis=1, keepdims=True)
    lse = m[:, 0] + jnp.log(jnp.sum(jnp.exp(sim - m), axis=1))
    ii = lax.broadcasted_iota(jnp.int32, (_BLK, _B), 0)
    jj = lax.broadcasted_iota(jnp.int32, (_BLK, _B), 1)
    diag = jnp.sum(jnp.where(jj == ii + k * _BLK, sim, 0.0), axis=1)
    con_ref[0, 0] += jnp.sum(lse - diag)


def _loss_call(ua, ia, u_emb, p_emb, n_emb, intents):
    grid = _B // _BLK
    scalar = jax.ShapeDtypeStruct((1, 1), jnp.float32)
    full = lambda shape: pl.BlockSpec(shape, lambda i: (0, 0))
    return pl.pallas_call(
        _loss_body,
        grid=(grid,),
        in_specs=[
            pl.BlockSpec((_BLK, _D), lambda i: (i, 0)),
            full((_B, _D)),
            full((_B, _D)),
            full((_B, _D)),
            full((_B, _D)),
            full((_D, _D)),
        ],
        out_specs=[pl.BlockSpec(memory_space=pltpu.SMEM)] * 3,
        out_shape=[scalar, scalar, scalar],
    )(ua, ia, u_emb, p_emb, n_emb, intents)


def _spmm(rows, cols, vals, mat, n_out):
    gathered = jnp.take(mat, cols, axis=0) * vals[:, None]
    return jax.ops.segment_sum(gathered, rows, num_segments=n_out)


def kernel(user_embedding, item_embedding, intents, W_w, W_b, u_mul_s,
           v_mul_s, ut, vt, adj_vals, adj_rows, adj_cols, user_ids,
           item_ids, pos_items, neg_items):
    n_users = user_embedding.shape[0]
    n_items = item_embedding.shape[0]
    ue, ie = user_embedding, item_embedding
    ue_sum, ie_sum = ue, ie
    msgu_sum = None
    msgi_sum = None
    n_layers = W_w.shape[0]
    for l in range(n_layers):
        mu = _spmm(adj_rows, adj_cols, adj_vals, ie, n_users)
        mi = _spmm(adj_cols, adj_rows, adj_vals, ue, n_items)
        msgu_sum = mu if msgu_sum is None else msgu_sum + mu
        msgi_sum = mi if msgi_sum is None else msgi_sum + mi
        b = W_b[l][None, :]
        ue_new = _layer_update(mu, ue, intents, W_w[l], b)
        ie_new = _layer_update(mi, ie, intents, W_w[l], b)
        ue, ie = ue_new, ie_new
        ue_sum = ue_sum + ue
        ie_sum = ie_sum + ie
    inv = 1.0 / (n_layers + 1)
    final_u = ue_sum * inv
    final_i = ie_sum * inv
    u_emb = jnp.take(final_u, user_ids, axis=0)
    p_emb = jnp.take(final_i, pos_items, axis=0)
    n_emb = jnp.take(final_i, neg_items, axis=0)
    ua = jnp.take(msgu_sum * (1.0 / n_layers), user_ids, axis=0)
    ia = jnp.take(msgi_sum * (1.0 / n_layers), item_ids, axis=0)
    bpr_s, con_s, reg_s = _loss_call(ua, ia, u_emb, p_emb, n_emb, intents)
    loss_bpr = bpr_s[0, 0] / _B
    loss_contrast = con_s[0, 0] / _B
    reg = _L3 * reg_s[0, 0]
    loss = _L1 * loss_bpr + _L2 * loss_contrast + reg
    return (loss, loss_bpr, loss_contrast, reg)
